# baseline (speedup 1.0000x reference)
import jax
import jax.numpy as jnp
from jax import lax
from jax.experimental import pallas as pl
from jax.experimental.pallas import tpu as pltpu
from jax.experimental.pallas import tpu_sc as plsc

OP_SHL = 0
OP_SHR = 1
MARK_AX = 2
ALU_LO = 16
ALU_HI = 32
AX_CARRY_LO = 48
OUTPUT_LO = 64
OUTPUT_HI = 80

B = 16384
D = 512
NC = 2
NS = 16
NW = NC * NS
ROWS_PER_W = B // NW
CHUNK = 64
NCHUNKS = ROWS_PER_W // CHUNK
NBUF = 3


def _patch_rows(buf, lanes):
    def row_body(r, carry):
        head = buf[r, pl.ds(OP_SHL, 16)]
        w_lo = buf[r, pl.ds(ALU_LO, 16)]
        w_hi = buf[r, pl.ds(ALU_HI, 16)]
        w_sh = buf[r, pl.ds(AX_CARRY_LO, 16)]

        def argmax16(w):
            return plsc.all_reduce_ffs(w == jnp.max(w, axis=0))

        a = argmax16(w_lo) + 16 * argmax16(w_hi)
        shv = jnp.minimum(argmax16(w_sh), 7)
        hi_shr = lax.shift_right_logical(a, shv + 4)

        hb = head > 0.5
        b_shl = plsc.all_reduce_population_count(hb & (lanes == OP_SHL)) > 0
        b_shr = plsc.all_reduce_population_count(hb & (lanes == OP_SHR)) > 0
        b_ax = plsc.all_reduce_population_count(hb & (lanes == MARK_AX)) > 0
        act_shl = b_shl & b_ax
        act = (b_shl | b_shr) & b_ax
        off = jnp.where(act_shl, 0, hi_shr)

        v_lo = buf[r, pl.ds(OUTPUT_LO, 16)]
        buf[r, pl.ds(OUTPUT_LO, 16)] = v_lo + jnp.where(
            act & (lanes == 0), 1.0, 0.0)
        v_hi = buf[r, pl.ds(OUTPUT_HI, 16)]
        buf[r, pl.ds(OUTPUT_HI, 16)] = v_hi + jnp.where(
            act & (lanes == off), 1.0, 0.0)
        return carry

    lax.fori_loop(0, CHUNK, row_body, 0)


def _sc_body(x_hbm, out_hbm, *scratch):
    bufs = scratch[:NBUF]
    sems_in = scratch[NBUF:2 * NBUF]
    sems_out = scratch[2 * NBUF:]
    wid = lax.axis_index("s") * NC + lax.axis_index("c")
    base = wid * ROWS_PER_W
    lanes = lax.iota(jnp.int32, 16)

    def start_in(t):
        return pltpu.async_copy(
            x_hbm.at[pl.ds(base + t * CHUNK, CHUNK)], bufs[t % NBUF],
            sems_in[t % NBUF])

    in_flight = {t: start_in(t) for t in range(min(2, NCHUNKS))}
    out_flight = {}

    for t in range(NCHUNKS):
        slot = t % NBUF
        buf = bufs[slot]
        in_flight.pop(t).wait()
        _patch_rows(buf, lanes)
        out_flight[t] = pltpu.async_copy(
            buf, out_hbm.at[pl.ds(base + t * CHUNK, CHUNK)], sems_out[slot])
        nxt = t + 2
        if nxt < NCHUNKS:
            prev = nxt - NBUF
            if prev in out_flight:
                out_flight.pop(prev).wait()
            in_flight[nxt] = start_in(nxt)
    for t in sorted(out_flight):
        out_flight[t].wait()


@jax.jit
def kernel(x):
    mesh = plsc.VectorSubcoreMesh(core_axis_name="c", subcore_axis_name="s")
    run = pl.kernel(
        _sc_body,
        out_type=jax.ShapeDtypeStruct((B, D), jnp.float32),
        mesh=mesh,
        scratch_types=(
            [pltpu.VMEM((CHUNK, D), jnp.float32)] * NBUF
            + [pltpu.SemaphoreType.DMA] * (2 * NBUF)
        ),
        compiler_params=pltpu.CompilerParams(
            use_tc_tiling_on_sc=True, needs_layout_passes=False),
    )
    return run(x)
